# baseline (speedup 1.0000x reference)
import functools

import jax
import jax.numpy as jnp
from jax import lax
from jax.experimental import pallas as pl
from jax.experimental.pallas import tpu as pltpu
from jax.experimental.pallas import tpu_sc as plsc

NC = 2
NS = 16
NW = NC * NS
CHUNK = 128


def _sc_gather_sum(nidx_t, features):
    k_samp, batch = nidx_t.shape
    feat_dim = features.shape[1]
    assert batch % CHUNK == 0
    n_chunks = batch // CHUNK
    per_w = (n_chunks + NW - 1) // NW
    mesh = plsc.VectorSubcoreMesh(
        core_axis_name="c", subcore_axis_name="s",
        num_cores=NC, num_subcores=NS)

    @functools.partial(
        pl.kernel,
        out_type=jax.ShapeDtypeStruct((batch, feat_dim), jnp.float32),
        mesh=mesh,
        scratch_types=[
            pltpu.VMEM((2, k_samp, CHUNK), jnp.int32),
            pltpu.VMEM((2, CHUNK, feat_dim), jnp.float32),
            pltpu.SemaphoreType.DMA,
            pltpu.SemaphoreType.DMA,
            pltpu.SemaphoreType.DMA,
            pltpu.SemaphoreType.DMA,
        ],
    )
    def sc_kernel(nidx_hbm, feat_hbm, out_hbm, idx2, acc2,
                  sem_idx, sem_j0, sem_add, sem_out):
        wid = lax.axis_index("s") * NC + lax.axis_index("c")

        def chunk_start(i):
            return pl.multiple_of((wid + i * NW) * CHUNK, CHUNK)

        def valid(i):
            return wid + i * NW < n_chunks

        def issue_idx(i, p):
            pltpu.async_copy(nidx_hbm.at[:, pl.ds(chunk_start(i), CHUNK)],
                             idx2.at[p], sem_idx)

        def wait_idx(p):
            pltpu.make_async_copy(nidx_hbm.at[:, pl.ds(0, CHUNK)],
                                  idx2.at[p], sem_idx).wait()

        def issue_j0(p):
            pltpu.async_copy(feat_hbm.at[idx2.at[p, 0]], acc2.at[p], sem_j0)

        def wait_j0(p):
            pltpu.make_async_copy(feat_hbm.at[pl.ds(0, CHUNK)], acc2.at[p],
                                  sem_j0).wait()

        def issue_adds(p):
            for _j in range(1, k_samp):
                pltpu.async_copy(feat_hbm.at[idx2.at[p, _j]], acc2.at[p],
                                 sem_add, add=True)

        def wait_adds(p):
            for _j in range(1, k_samp):
                pltpu.make_async_copy(feat_hbm.at[pl.ds(0, CHUNK)],
                                      acc2.at[p], sem_add).wait()

        def issue_out(i, p):
            pltpu.async_copy(acc2.at[p],
                             out_hbm.at[pl.ds(chunk_start(i), CHUNK), :],
                             sem_out)

        def wait_out(p):
            pltpu.make_async_copy(acc2.at[p],
                                  out_hbm.at[pl.ds(0, CHUNK), :],
                                  sem_out).wait()

        issue_idx(0, 0)

        @pl.when(valid(1))
        def _():
            issue_idx(1, 1)

        wait_idx(0)
        issue_j0(0)

        def body(i, carry):
            p = jnp.bitwise_and(i, 1)

            @pl.when(valid(i))
            def _():
                wait_j0(p)
                issue_adds(p)

                @pl.when(valid(i + 1))
                def _():
                    wait_idx(1 - p)

                    @pl.when(i >= 1)
                    def _():
                        wait_out(1 - p)

                    issue_j0(1 - p)

                wait_adds(p)
                issue_out(i, p)

                @pl.when(valid(i + 2))
                def _():
                    issue_idx(i + 2, p)

            return carry

        lax.fori_loop(0, per_w, body, 0)
        wait_out(0)
        wait_out(1)

    return sc_kernel(nidx_t, features)


def _tc_mlp(s, w0t, w1t, rows, total, out_buf, blk_off):
    feat_dim = s.shape[1]
    blk = 1000

    def body(s_ref, w0_ref, w1_ref, *rest):
        o_ref = rest[-1]
        h = jnp.dot(s_ref[...], w0_ref[...],
                    preferred_element_type=jnp.float32)
        h = jnp.maximum(h, 0.0)
        o = jnp.dot(h, w1_ref[...], preferred_element_type=jnp.float32)
        o_ref[...] = jnp.maximum(o, 0.0)

    in_specs = [
        pl.BlockSpec((blk, feat_dim), lambda i: (i, 0)),
        pl.BlockSpec(w0t.shape, lambda i: (0, 0)),
        pl.BlockSpec(w1t.shape, lambda i: (0, 0)),
    ]
    args = [s, w0t, w1t]
    aliases = {}
    if out_buf is not None:
        in_specs.append(pl.BlockSpec(memory_space=pl.ANY))
        args.append(out_buf)
        aliases = {3: 0}

    return pl.pallas_call(
        body,
        grid=(rows // blk,),
        in_specs=in_specs,
        out_specs=pl.BlockSpec((blk, feat_dim),
                               lambda i: (i + blk_off, 0)),
        out_shape=jax.ShapeDtypeStruct((total, feat_dim), jnp.float32),
        input_output_aliases=aliases,
    )(*args)


def kernel(nodes, neigh_idx, features, W, fc_W):
    batch, k_samp = neigh_idx.shape
    w0t = W.T * (1.0 / k_samp)
    w1t = fc_W.T
    half = batch // 2
    pad = (-half) % CHUNK
    s0 = _sc_gather_sum(jnp.pad(neigh_idx[:half], ((0, pad), (0, 0))).T,
                        features)
    s1 = _sc_gather_sum(jnp.pad(neigh_idx[half:], ((0, pad), (0, 0))).T,
                        features)
    out = _tc_mlp(s0, w0t, w1t, half, batch, None, 0)
    return _tc_mlp(s1, w0t, w1t, half, batch, out, half // 1000)


# speedup vs baseline: 1.8890x; 1.8890x over previous
import functools

import jax
import jax.numpy as jnp
from jax import lax
from jax.experimental import pallas as pl
from jax.experimental.pallas import tpu as pltpu
from jax.experimental.pallas import tpu_sc as plsc

NC = 2
NS = 16
NW = NC * NS
CHUNK = 128


def _sc_gather_sum(nidx_t, features):
    k_samp, batch = nidx_t.shape
    feat_dim = features.shape[1]
    assert batch % CHUNK == 0
    n_chunks = batch // CHUNK
    per_w = (n_chunks + NW - 1) // NW
    mesh = plsc.VectorSubcoreMesh(
        core_axis_name="c", subcore_axis_name="s",
        num_cores=NC, num_subcores=NS)

    @functools.partial(
        pl.kernel,
        out_type=jax.ShapeDtypeStruct((batch, feat_dim), jnp.float32),
        mesh=mesh,
        scratch_types=[
            pltpu.VMEM((2, k_samp, CHUNK), jnp.int32),
            pltpu.VMEM((2, CHUNK, feat_dim), jnp.float32),
            pltpu.SemaphoreType.DMA,
            pltpu.SemaphoreType.DMA,
            pltpu.SemaphoreType.DMA,
            pltpu.SemaphoreType.DMA,
        ],
    )
    def sc_kernel(nidx_hbm, feat_hbm, out_hbm, idx2, acc2,
                  sem_idx, sem_j0, sem_add, sem_out):
        wid = lax.axis_index("s") * NC + lax.axis_index("c")

        def chunk_start(i):
            return pl.multiple_of((wid + i * NW) * CHUNK, CHUNK)

        def valid(i):
            return wid + i * NW < n_chunks

        def issue_idx(i, p):
            pltpu.async_copy(nidx_hbm.at[:, pl.ds(chunk_start(i), CHUNK)],
                             idx2.at[p], sem_idx)

        def wait_idx(p):
            pltpu.make_async_copy(nidx_hbm.at[:, pl.ds(0, CHUNK)],
                                  idx2.at[p], sem_idx).wait()

        def issue_j0(p):
            pltpu.async_copy(feat_hbm.at[idx2.at[p, 0]], acc2.at[p], sem_j0)

        def wait_j0(p):
            pltpu.make_async_copy(feat_hbm.at[pl.ds(0, CHUNK)], acc2.at[p],
                                  sem_j0).wait()

        def issue_adds(p):
            for _j in range(1, k_samp):
                pltpu.async_copy(feat_hbm.at[idx2.at[p, _j]], acc2.at[p],
                                 sem_add, add=True)

        def wait_adds(p):
            for _j in range(1, k_samp):
                pltpu.make_async_copy(feat_hbm.at[pl.ds(0, CHUNK)],
                                      acc2.at[p], sem_add).wait()

        def issue_out(i, p):
            pltpu.async_copy(acc2.at[p],
                             out_hbm.at[pl.ds(chunk_start(i), CHUNK), :],
                             sem_out)

        def wait_out(p):
            pltpu.make_async_copy(acc2.at[p],
                                  out_hbm.at[pl.ds(0, CHUNK), :],
                                  sem_out).wait()

        issue_idx(0, 0)

        @pl.when(valid(1))
        def _():
            issue_idx(1, 1)

        wait_idx(0)
        issue_j0(0)

        def body(i, carry):
            p = jnp.bitwise_and(i, 1)

            @pl.when(valid(i))
            def _():
                wait_j0(p)
                issue_adds(p)

                @pl.when(valid(i + 1))
                def _():
                    wait_idx(1 - p)

                    @pl.when(i >= 1)
                    def _():
                        wait_out(1 - p)

                    issue_j0(1 - p)

                wait_adds(p)
                issue_out(i, p)

                @pl.when(valid(i + 2))
                def _():
                    issue_idx(i + 2, p)

            return carry

        lax.fori_loop(0, per_w, body, 0)
        wait_out(0)
        wait_out(1)

    return sc_kernel(nidx_t, features)


def _tc_mlp(s, w0t, w1t, rows, total, out_buf, blk_off):
    feat_dim = s.shape[1]
    blk = 5000

    def body(s_ref, w0_ref, w1_ref, *rest):
        o_ref = rest[-1]
        h = jnp.dot(s_ref[...], w0_ref[...],
                    preferred_element_type=jnp.float32)
        h = jnp.maximum(h, 0.0)
        o = jnp.dot(h, w1_ref[...], preferred_element_type=jnp.float32)
        o_ref[...] = jnp.maximum(o, 0.0)

    in_specs = [
        pl.BlockSpec((blk, feat_dim), lambda i: (i, 0)),
        pl.BlockSpec(w0t.shape, lambda i: (0, 0)),
        pl.BlockSpec(w1t.shape, lambda i: (0, 0)),
    ]
    args = [s, w0t, w1t]
    aliases = {}
    if out_buf is not None:
        in_specs.append(pl.BlockSpec(memory_space=pl.ANY))
        args.append(out_buf)
        aliases = {3: 0}

    return pl.pallas_call(
        body,
        grid=(rows // blk,),
        in_specs=in_specs,
        out_specs=pl.BlockSpec((blk, feat_dim),
                               lambda i: (i + blk_off, 0)),
        out_shape=jax.ShapeDtypeStruct((total, feat_dim), jnp.float32),
        input_output_aliases=aliases,
    )(*args)


def kernel(nodes, neigh_idx, features, W, fc_W):
    batch, k_samp = neigh_idx.shape
    w0t = W.T * (1.0 / k_samp)
    w1t = fc_W.T
    pad = (-batch) % CHUNK
    nidx_t = jnp.pad(neigh_idx, ((0, pad), (0, 0))).T
    s = _sc_gather_sum(nidx_t, features)
    return _tc_mlp(s, w0t, w1t, batch, batch, None, 0)


# speedup vs baseline: 1.9321x; 1.0228x over previous
import functools

import jax
import jax.numpy as jnp
from jax import lax
from jax.experimental import pallas as pl
from jax.experimental.pallas import tpu as pltpu
from jax.experimental.pallas import tpu_sc as plsc

NC = 2
NS = 16
NW = NC * NS
CHUNK = 128


def _sc_gather_sum(nidx_t, features):
    k_samp, batch = nidx_t.shape
    feat_dim = features.shape[1]
    assert batch % CHUNK == 0
    n_chunks = batch // CHUNK
    per_w = (n_chunks + NW - 1) // NW
    mesh = plsc.VectorSubcoreMesh(
        core_axis_name="c", subcore_axis_name="s",
        num_cores=NC, num_subcores=NS)

    @functools.partial(
        pl.kernel,
        out_type=jax.ShapeDtypeStruct((batch, feat_dim), jnp.float32),
        mesh=mesh,
        scratch_types=[
            pltpu.VMEM((2, k_samp, CHUNK), jnp.int32),
            pltpu.VMEM((2, CHUNK, feat_dim), jnp.float32),
            pltpu.SemaphoreType.DMA,
            pltpu.SemaphoreType.DMA,
            pltpu.SemaphoreType.DMA,
            pltpu.SemaphoreType.DMA,
        ],
    )
    def sc_kernel(nidx_hbm, feat_hbm, out_hbm, idx2, acc2,
                  sem_idx, sem_j0, sem_add, sem_out):
        wid = lax.axis_index("s") * NC + lax.axis_index("c")

        def chunk_start(i):
            return pl.multiple_of((wid + i * NW) * CHUNK, CHUNK)

        def valid(i):
            return wid + i * NW < n_chunks

        def issue_idx(i, p):
            pltpu.async_copy(nidx_hbm.at[:, pl.ds(chunk_start(i), CHUNK)],
                             idx2.at[p], sem_idx)

        def wait_idx(p):
            pltpu.make_async_copy(nidx_hbm.at[:, pl.ds(0, CHUNK)],
                                  idx2.at[p], sem_idx).wait()

        def issue_j0(p):
            pltpu.async_copy(feat_hbm.at[idx2.at[p, 0]], acc2.at[p], sem_j0)

        def wait_j0(p):
            pltpu.make_async_copy(feat_hbm.at[pl.ds(0, CHUNK)], acc2.at[p],
                                  sem_j0).wait()

        def issue_adds(p):
            for _j in range(1, k_samp):
                pltpu.async_copy(feat_hbm.at[idx2.at[p, _j]], acc2.at[p],
                                 sem_add, add=True)

        def wait_adds(p):
            for _j in range(1, k_samp):
                pltpu.make_async_copy(feat_hbm.at[pl.ds(0, CHUNK)],
                                      acc2.at[p], sem_add).wait()

        def issue_out(i, p):
            pltpu.async_copy(acc2.at[p],
                             out_hbm.at[pl.ds(chunk_start(i), CHUNK), :],
                             sem_out)

        def wait_out(p):
            pltpu.make_async_copy(acc2.at[p],
                                  out_hbm.at[pl.ds(0, CHUNK), :],
                                  sem_out).wait()

        issue_idx(0, 0)

        @pl.when(valid(1))
        def _():
            issue_idx(1, 1)

        wait_idx(0)
        issue_j0(0)

        def body(i, carry):
            p = jnp.bitwise_and(i, 1)

            @pl.when(valid(i))
            def _():
                wait_j0(p)
                issue_adds(p)

                @pl.when(valid(i + 1))
                def _():
                    wait_idx(1 - p)

                    @pl.when(i >= 1)
                    def _():
                        wait_out(1 - p)

                    issue_j0(1 - p)

                wait_adds(p)
                issue_out(i, p)

                @pl.when(valid(i + 2))
                def _():
                    issue_idx(i + 2, p)

            return carry

        lax.fori_loop(0, per_w, body, 0)
        wait_out(0)
        wait_out(1)

    return sc_kernel(nidx_t, features)


def _tc_mlp(s, w0t, w1t, rows, total, out_buf, blk_off):
    feat_dim = s.shape[1]
    blk = 10000

    def body(s_ref, w0_ref, w1_ref, *rest):
        o_ref = rest[-1]
        h = jnp.dot(s_ref[...], w0_ref[...],
                    preferred_element_type=jnp.float32)
        h = jnp.maximum(h, 0.0)
        o = jnp.dot(h, w1_ref[...], preferred_element_type=jnp.float32)
        o_ref[...] = jnp.maximum(o, 0.0)

    in_specs = [
        pl.BlockSpec((blk, feat_dim), lambda i: (i, 0)),
        pl.BlockSpec(w0t.shape, lambda i: (0, 0)),
        pl.BlockSpec(w1t.shape, lambda i: (0, 0)),
    ]
    args = [s, w0t, w1t]
    aliases = {}
    if out_buf is not None:
        in_specs.append(pl.BlockSpec(memory_space=pl.ANY))
        args.append(out_buf)
        aliases = {3: 0}

    return pl.pallas_call(
        body,
        grid=(rows // blk,),
        in_specs=in_specs,
        out_specs=pl.BlockSpec((blk, feat_dim),
                               lambda i: (i + blk_off, 0)),
        out_shape=jax.ShapeDtypeStruct((total, feat_dim), jnp.float32),
        input_output_aliases=aliases,
    )(*args)


def kernel(nodes, neigh_idx, features, W, fc_W):
    batch, k_samp = neigh_idx.shape
    w0t = W.T * (1.0 / k_samp)
    w1t = fc_W.T
    pad = (-batch) % CHUNK
    nidx_t = jnp.pad(neigh_idx, ((0, pad), (0, 0))).T
    s = _sc_gather_sum(nidx_t, features)
    return _tc_mlp(s, w0t, w1t, batch, batch, None, 0)
